# baseline (speedup 1.0000x reference)
import jax
import jax.numpy as jnp
from jax import lax
from jax.experimental import pallas as pl
from jax.experimental.pallas import tpu as pltpu
from jax.experimental.pallas import tpu_sc as plsc

N = 10000
D = 128
HID = 512
E = 320000
K = 10

NC = 2
NS = 16
NW = NC * NS
BE = 96
NSEG = 3
NBS = 36
NB = NSEG * NBS
EPT = NB * BE
EPAD = NW * EPT
NP = 10240
RPT = NP // NS

_MESH = plsc.VectorSubcoreMesh(core_axis_name="c", subcore_axis_name="s")


def _hop_body(
    g_hbm, srcT, dstT, out,
    idxs, idxd, rows0, rows1, rows2, acc,
    sg0, sg1, sg2, ss0, ss1, ss2,
):
    c = lax.axis_index("c")
    s_ = lax.axis_index("s")
    wid = c * NS + s_
    rows = (rows0, rows1, rows2)
    sgs = (sg0, sg1, sg2)
    sss = (ss0, ss1, ss2)

    z = jnp.zeros((16,), jnp.float32)

    @pl.loop(0, BE)
    def _(i):
        for j in range(D // 16):
            rows0[i, pl.ds(j * 16, 16)] = z

    nz = RPT // BE
    rem = RPT - nz * BE
    for t in range(nz):
        pltpu.async_copy(
            rows0, acc.at[pl.ds(s_ * RPT + t * BE, BE)], sgs[t % 3]
        )
    if rem:
        pltpu.async_copy(
            rows0.at[pl.ds(0, rem)],
            acc.at[pl.ds(s_ * RPT + nz * BE, rem)],
            sss[0],
        )
    for t in range(nz):
        pltpu.make_async_copy(
            rows0, acc.at[pl.ds(s_ * RPT + t * BE, BE)], sgs[t % 3]
        ).wait()
    if rem:
        pltpu.make_async_copy(
            rows0.at[pl.ds(0, rem)],
            acc.at[pl.ds(s_ * RPT + nz * BE, rem)],
            sss[0],
        ).wait()
    plsc.subcore_barrier()

    for seg in range(NSEG):
        pltpu.sync_copy(srcT.at[wid, seg], idxs)
        pltpu.sync_copy(dstT.at[wid, seg], idxd)

        for b in range(3):
            pltpu.async_copy(g_hbm.at[idxs.at[b]], rows[b], sgs[b])

        @pl.loop(0, NBS // 3 - 1)
        def _(t):
            j = 3 * t
            for b in range(3):
                pltpu.make_async_copy(g_hbm.at[idxs.at[j + b]], rows[b], sgs[b]).wait()
                pltpu.async_copy(rows[b], acc.at[idxd.at[j + b]], sss[b], add=True)
            for b in range(3):
                pltpu.make_async_copy(rows[b], acc.at[idxd.at[j + b]], sss[b]).wait()
                pltpu.async_copy(g_hbm.at[idxs.at[j + 3 + b]], rows[b], sgs[b])

        jlast = NBS - 3
        for b in range(3):
            pltpu.make_async_copy(g_hbm.at[idxs.at[jlast + b]], rows[b], sgs[b]).wait()
            pltpu.async_copy(rows[b], acc.at[idxd.at[jlast + b]], sss[b], add=True)
        for b in range(3):
            pltpu.make_async_copy(rows[b], acc.at[idxd.at[jlast + b]], sss[b]).wait()

    plsc.subcore_barrier()
    pltpu.sync_copy(acc.at[pl.ds(s_ * RPT, RPT)], out.at[c, pl.ds(s_ * RPT, RPT)])


_hop_kernel = pl.kernel(
    _hop_body,
    out_type=jax.ShapeDtypeStruct((NC, NP, D), jnp.float32),
    mesh=_MESH,
    scratch_types=[
        pltpu.VMEM((NBS, BE), jnp.int32),
        pltpu.VMEM((NBS, BE), jnp.int32),
        pltpu.VMEM((BE, D), jnp.float32),
        pltpu.VMEM((BE, D), jnp.float32),
        pltpu.VMEM((BE, D), jnp.float32),
        pltpu.VMEM_SHARED((NP, D), jnp.float32),
        pltpu.SemaphoreType.DMA,
        pltpu.SemaphoreType.DMA,
        pltpu.SemaphoreType.DMA,
        pltpu.SemaphoreType.DMA,
        pltpu.SemaphoreType.DMA,
        pltpu.SemaphoreType.DMA,
    ],
)


BN = 1000


def _mlp_body(x_ref, w1_ref, b1_ref, w2_ref, b2_ref, o_ref):
    h = jnp.dot(x_ref[...], w1_ref[...], preferred_element_type=jnp.float32)
    h = jnp.maximum(h + b1_ref[...][None, :], 0.0)
    o_ref[...] = (
        jnp.dot(h, w2_ref[...], preferred_element_type=jnp.float32)
        + b2_ref[...][None, :]
    )


_mlp_kernel = pl.pallas_call(
    _mlp_body,
    grid=(N // BN,),
    in_specs=[
        pl.BlockSpec((BN, D), lambda i: (i, 0)),
        pl.BlockSpec((D, HID), lambda i: (0, 0)),
        pl.BlockSpec((HID,), lambda i: (0,)),
        pl.BlockSpec((HID, D), lambda i: (0, 0)),
        pl.BlockSpec((D,), lambda i: (0,)),
    ],
    out_specs=pl.BlockSpec((BN, D), lambda i: (i, 0)),
    out_shape=jax.ShapeDtypeStruct((N, D), jnp.float32),
)


def _prep_body(degp_ref, h0_ref, norm2_ref, rnorm_ref, g0_ref):
    deg = degp_ref[0, :, 0:1] + degp_ref[1, :, 0:1]
    norm = lax.rsqrt(deg)
    norm2_ref[...] = 1.0 / deg
    rnorm_ref[...] = jnp.sqrt(deg)
    g0_ref[...] = h0_ref[...] * norm


_prep_kernel = pl.pallas_call(
    _prep_body,
    grid=(N // BN,),
    in_specs=[
        pl.BlockSpec((NC, BN, D), lambda i: (0, i, 0)),
        pl.BlockSpec((BN, D), lambda i: (i, 0)),
    ],
    out_specs=[
        pl.BlockSpec((BN, 1), lambda i: (i, 0)),
        pl.BlockSpec((BN, 1), lambda i: (i, 0)),
        pl.BlockSpec((BN, D), lambda i: (i, 0)),
    ],
    out_shape=[
        jax.ShapeDtypeStruct((N, 1), jnp.float32),
        jax.ShapeDtypeStruct((N, 1), jnp.float32),
        jax.ShapeDtypeStruct((N, D), jnp.float32),
    ],
)


def _comb_body(p_ref, norm2_ref, g_ref):
    g_ref[...] = (p_ref[0] + p_ref[1]) * norm2_ref[...]


_comb_kernel = pl.pallas_call(
    _comb_body,
    grid=(N // BN,),
    in_specs=[
        pl.BlockSpec((NC, BN, D), lambda i: (0, i, 0)),
        pl.BlockSpec((BN, 1), lambda i: (i, 0)),
    ],
    out_specs=pl.BlockSpec((BN, D), lambda i: (i, 0)),
    out_shape=jax.ShapeDtypeStruct((N, D), jnp.float32),
)


def _final_body(s_ref, rnorm_ref, h0_ref, *refs):
    g_refs = refs[:K]
    o_ref = refs[K]
    sb = s_ref[...]
    rn = rnorm_ref[...]
    acc = jnp.zeros((BN, D), jnp.float32)
    for k in range(K + 1):
        hk = h0_ref[...] if k == 0 else g_refs[k - 1][...] * rn
        z = jnp.dot(hk, sb, preferred_element_type=jnp.float32)
        acc = acc + jax.nn.sigmoid(z) * hk
    o_ref[...] = acc


_final_kernel = pl.pallas_call(
    _final_body,
    grid=(N // BN,),
    in_specs=[
        pl.BlockSpec((D, 1), lambda i: (0, 0)),
        pl.BlockSpec((BN, 1), lambda i: (i, 0)),
        pl.BlockSpec((BN, D), lambda i: (i, 0)),
    ]
    + [pl.BlockSpec((BN, D), lambda i: (i, 0)) for _ in range(K)],
    out_specs=pl.BlockSpec((BN, D), lambda i: (i, 0)),
    out_shape=jax.ShapeDtypeStruct((N, D), jnp.float32),
)


@jax.jit
def kernel(feats, edge_index, W1, b1, W2, b2, s):
    npad = EPAD - E
    src = jnp.concatenate(
        [
            edge_index[0].astype(jnp.int32),
            (jnp.arange(npad, dtype=jnp.int32) * 41) % N,
        ]
    ).reshape(NW, NSEG, NBS, BE)
    dst = jnp.concatenate(
        [
            edge_index[1].astype(jnp.int32),
            N + (jnp.arange(npad, dtype=jnp.int32) % (NP - N)),
        ]
    ).reshape(NW, NSEG, NBS, BE)

    h0 = _mlp_kernel(feats, W1, b1, W2, b2)
    degp = _hop_kernel(jnp.ones((N, D), jnp.float32), src, dst)
    norm2, rnorm, g = _prep_kernel(degp, h0)

    gs = []
    for _ in range(K):
        p = _hop_kernel(g, src, dst)
        g = _comb_kernel(p, norm2)
        gs.append(g)

    return _final_kernel(s, rnorm, h0, *gs)


# speedup vs baseline: 1.0342x; 1.0342x over previous
import jax
import jax.numpy as jnp
from jax import lax
from jax.experimental import pallas as pl
from jax.experimental.pallas import tpu as pltpu
from jax.experimental.pallas import tpu_sc as plsc

N = 10000
D = 128
HID = 512
E = 320000
K = 10

NC = 2
NS = 16
NW = NC * NS
BE = 80
NSEG = 2
NBS = 63
NB = NSEG * NBS
EPT = NB * BE
EPAD = NW * EPT
NP = 10240
RPT = NP // NS

_MESH = plsc.VectorSubcoreMesh(core_axis_name="c", subcore_axis_name="s")


def _hop_body(
    g_hbm, srcT, dstT, out,
    idxs, idxd, rows0, rows1, rows2, acc,
    sg0, sg1, sg2, ss0, ss1, ss2,
):
    c = lax.axis_index("c")
    s_ = lax.axis_index("s")
    wid = c * NS + s_
    rows = (rows0, rows1, rows2)
    sgs = (sg0, sg1, sg2)
    sss = (ss0, ss1, ss2)

    z = jnp.zeros((16,), jnp.float32)

    @pl.loop(0, BE)
    def _(i):
        for j in range(D // 16):
            rows0[i, pl.ds(j * 16, 16)] = z

    nz = RPT // BE
    rem = RPT - nz * BE
    for t in range(nz):
        pltpu.async_copy(
            rows0, acc.at[pl.ds(s_ * RPT + t * BE, BE)], sgs[t % 3]
        )
    if rem:
        pltpu.async_copy(
            rows0.at[pl.ds(0, rem)],
            acc.at[pl.ds(s_ * RPT + nz * BE, rem)],
            sss[0],
        )
    for t in range(nz):
        pltpu.make_async_copy(
            rows0, acc.at[pl.ds(s_ * RPT + t * BE, BE)], sgs[t % 3]
        ).wait()
    if rem:
        pltpu.make_async_copy(
            rows0.at[pl.ds(0, rem)],
            acc.at[pl.ds(s_ * RPT + nz * BE, rem)],
            sss[0],
        ).wait()
    plsc.subcore_barrier()

    for seg in range(NSEG):
        pltpu.sync_copy(srcT.at[wid, seg], idxs)
        pltpu.sync_copy(dstT.at[wid, seg], idxd)

        for b in range(3):
            pltpu.async_copy(g_hbm.at[idxs.at[b]], rows[b], sgs[b])

        @pl.loop(0, NBS // 3 - 1)
        def _(t):
            j = 3 * t
            for b in range(3):
                pltpu.make_async_copy(g_hbm.at[idxs.at[j + b]], rows[b], sgs[b]).wait()
                pltpu.async_copy(rows[b], acc.at[idxd.at[j + b]], sss[b], add=True)
            for b in range(3):
                pltpu.make_async_copy(rows[b], acc.at[idxd.at[j + b]], sss[b]).wait()
                pltpu.async_copy(g_hbm.at[idxs.at[j + 3 + b]], rows[b], sgs[b])

        jlast = NBS - 3
        for b in range(3):
            pltpu.make_async_copy(g_hbm.at[idxs.at[jlast + b]], rows[b], sgs[b]).wait()
            pltpu.async_copy(rows[b], acc.at[idxd.at[jlast + b]], sss[b], add=True)
        for b in range(3):
            pltpu.make_async_copy(rows[b], acc.at[idxd.at[jlast + b]], sss[b]).wait()

    plsc.subcore_barrier()
    pltpu.sync_copy(acc.at[pl.ds(s_ * RPT, RPT)], out.at[c, pl.ds(s_ * RPT, RPT)])


_hop_kernel = pl.kernel(
    _hop_body,
    out_type=jax.ShapeDtypeStruct((NC, NP, D), jnp.float32),
    mesh=_MESH,
    scratch_types=[
        pltpu.VMEM((NBS, BE), jnp.int32),
        pltpu.VMEM((NBS, BE), jnp.int32),
        pltpu.VMEM((BE, D), jnp.float32),
        pltpu.VMEM((BE, D), jnp.float32),
        pltpu.VMEM((BE, D), jnp.float32),
        pltpu.VMEM_SHARED((NP, D), jnp.float32),
        pltpu.SemaphoreType.DMA,
        pltpu.SemaphoreType.DMA,
        pltpu.SemaphoreType.DMA,
        pltpu.SemaphoreType.DMA,
        pltpu.SemaphoreType.DMA,
        pltpu.SemaphoreType.DMA,
    ],
)


BN = 1000


def _mlp_body(x_ref, w1_ref, b1_ref, w2_ref, b2_ref, o_ref):
    h = jnp.dot(x_ref[...], w1_ref[...], preferred_element_type=jnp.float32)
    h = jnp.maximum(h + b1_ref[...][None, :], 0.0)
    o_ref[...] = (
        jnp.dot(h, w2_ref[...], preferred_element_type=jnp.float32)
        + b2_ref[...][None, :]
    )


_mlp_kernel = pl.pallas_call(
    _mlp_body,
    grid=(N // BN,),
    in_specs=[
        pl.BlockSpec((BN, D), lambda i: (i, 0)),
        pl.BlockSpec((D, HID), lambda i: (0, 0)),
        pl.BlockSpec((HID,), lambda i: (0,)),
        pl.BlockSpec((HID, D), lambda i: (0, 0)),
        pl.BlockSpec((D,), lambda i: (0,)),
    ],
    out_specs=pl.BlockSpec((BN, D), lambda i: (i, 0)),
    out_shape=jax.ShapeDtypeStruct((N, D), jnp.float32),
)


def _prep_body(degp_ref, h0_ref, norm2_ref, rnorm_ref, g0_ref):
    deg = degp_ref[0, :, 0:1] + degp_ref[1, :, 0:1]
    norm = lax.rsqrt(deg)
    norm2_ref[...] = 1.0 / deg
    rnorm_ref[...] = jnp.sqrt(deg)
    g0_ref[...] = h0_ref[...] * norm


_prep_kernel = pl.pallas_call(
    _prep_body,
    grid=(N // BN,),
    in_specs=[
        pl.BlockSpec((NC, BN, D), lambda i: (0, i, 0)),
        pl.BlockSpec((BN, D), lambda i: (i, 0)),
    ],
    out_specs=[
        pl.BlockSpec((BN, 1), lambda i: (i, 0)),
        pl.BlockSpec((BN, 1), lambda i: (i, 0)),
        pl.BlockSpec((BN, D), lambda i: (i, 0)),
    ],
    out_shape=[
        jax.ShapeDtypeStruct((N, 1), jnp.float32),
        jax.ShapeDtypeStruct((N, 1), jnp.float32),
        jax.ShapeDtypeStruct((N, D), jnp.float32),
    ],
)


def _comb_body(p_ref, norm2_ref, g_ref):
    g_ref[...] = (p_ref[0] + p_ref[1]) * norm2_ref[...]


_comb_kernel = pl.pallas_call(
    _comb_body,
    grid=(N // BN,),
    in_specs=[
        pl.BlockSpec((NC, BN, D), lambda i: (0, i, 0)),
        pl.BlockSpec((BN, 1), lambda i: (i, 0)),
    ],
    out_specs=pl.BlockSpec((BN, D), lambda i: (i, 0)),
    out_shape=jax.ShapeDtypeStruct((N, D), jnp.float32),
)


def _final_body(s_ref, rnorm_ref, h0_ref, *refs):
    g_refs = refs[:K]
    o_ref = refs[K]
    sb = s_ref[...]
    rn = rnorm_ref[...]
    acc = jnp.zeros((BN, D), jnp.float32)
    for k in range(K + 1):
        hk = h0_ref[...] if k == 0 else g_refs[k - 1][...] * rn
        z = jnp.dot(hk, sb, preferred_element_type=jnp.float32)
        acc = acc + jax.nn.sigmoid(z) * hk
    o_ref[...] = acc


_final_kernel = pl.pallas_call(
    _final_body,
    grid=(N // BN,),
    in_specs=[
        pl.BlockSpec((D, 1), lambda i: (0, 0)),
        pl.BlockSpec((BN, 1), lambda i: (i, 0)),
        pl.BlockSpec((BN, D), lambda i: (i, 0)),
    ]
    + [pl.BlockSpec((BN, D), lambda i: (i, 0)) for _ in range(K)],
    out_specs=pl.BlockSpec((BN, D), lambda i: (i, 0)),
    out_shape=jax.ShapeDtypeStruct((N, D), jnp.float32),
)


@jax.jit
def kernel(feats, edge_index, W1, b1, W2, b2, s):
    npad = EPAD - E
    src = jnp.concatenate(
        [
            edge_index[0].astype(jnp.int32),
            (jnp.arange(npad, dtype=jnp.int32) * 41) % N,
        ]
    ).reshape(NW, NSEG, NBS, BE)
    dst = jnp.concatenate(
        [
            edge_index[1].astype(jnp.int32),
            N + (jnp.arange(npad, dtype=jnp.int32) % (NP - N)),
        ]
    ).reshape(NW, NSEG, NBS, BE)

    h0 = _mlp_kernel(feats, W1, b1, W2, b2)
    degp = _hop_kernel(jnp.ones((N, D), jnp.float32), src, dst)
    norm2, rnorm, g = _prep_kernel(degp, h0)

    gs = []
    for _ in range(K):
        p = _hop_kernel(g, src, dst)
        g = _comb_kernel(p, norm2)
        gs.append(g)

    return _final_kernel(s, rnorm, h0, *gs)
